# Initial kernel scaffold; baseline (speedup 1.0000x reference)
import jax
import jax.numpy as jnp
from jax.experimental import pallas as pl


def kernel(state24, Wl1, bl1, Wr1, br1, att1, bias1, Wl2, bl2, Wr2, br2, att2, bias2, W1, b1, W2, b2, W3, b3):
    raise NotImplementedError("write your pallas kernel here")


# speedup vs baseline: 557.0808x; 557.0808x over previous
import jax
import jax.numpy as jnp
import numpy as np
from jax.experimental import pallas as pl

N_NODES = 21
B = 1024
HALF = B // 2
NHALF = HALF * N_NODES
MAX_RANGE = 10.0
TILE = 128
ROWS = TILE * N_NODES
GRID = HALF // TILE

_HIGH = jax.lax.Precision.HIGHEST


def _angle_feat_np():
    bound = np.linspace(-np.pi / 2 - 0.03, np.pi / 2, 21)[:-1]
    angles = bound + np.pi / 20
    return np.stack([np.sin(angles), np.cos(angles)], axis=1).astype(np.float32)


def _head_selector(att_shape_heads, ch):
    c = np.arange(att_shape_heads * ch)
    m = (c[:, None] // ch == np.arange(att_shape_heads)[None, :]).astype(np.float32)
    return m


_E1_np = _head_selector(4, 64).T
_POOL_np = np.kron(np.eye(TILE, dtype=np.float32),
                   np.full((1, N_NODES), 1.0 / N_NODES, dtype=np.float32))
_ANGLE_np = _angle_feat_np()


def _dot(a, b):
    return jnp.dot(a, b, precision=_HIGH, preferred_element_type=jnp.float32)


def _leaky(x):
    return jnp.where(x > 0, x, 0.2 * x)


def _elu(x):
    return jnp.where(x > 0, x, jnp.exp(jnp.minimum(x, 0.0)) - 1.0)


def _actor_kernel(xa_ref, xb_ref,
                  wl1_ref, bl1_ref, wr1_ref, br1_ref, a1_ref, e1_ref, bias1_ref,
                  wl2_ref, bl2_ref, wr2_ref, br2_ref, att2_ref, bias2_ref,
                  pool_ref,
                  w1_ref, b1_ref, w2_ref, b2_ref, w3_ref, b3_ref,
                  outa_ref, outb_ref):
    xa = xa_ref[...]
    xb = xb_ref[...]

    wl1 = wl1_ref[...]
    bl1 = bl1_ref[...]
    xlA = _dot(xa, wl1) + bl1
    xlB = _dot(xb, wl1) + bl1
    xrB = _dot(xb, wr1_ref[...]) + br1_ref[...]

    a1 = a1_ref[...]
    a_pair = _dot(_leaky(xrB + xlA), a1)
    a_self = _dot(_leaky(xrB + xlB), a1)
    m = jnp.maximum(a_pair, a_self)
    wp = 42.0 * jnp.exp(a_pair - m)
    ws = jnp.exp(a_self - m)
    inv = 1.0 / (wp + ws + 1e-16)
    e1 = e1_ref[...]
    cp = _dot(wp * inv, e1)
    cs = _dot(ws * inv, e1)
    bias1 = bias1_ref[...]
    h1A = _elu(xlA + bias1)
    h1B = _elu(cp * xlA + cs * xlB + bias1)

    wl2 = wl2_ref[...]
    bl2 = bl2_ref[...]
    xl2A = _dot(h1A, wl2) + bl2
    xl2B = _dot(h1B, wl2) + bl2
    xr2B = _dot(h1B, wr2_ref[...]) + br2_ref[...]

    att2 = att2_ref[...]
    a_pair2 = jnp.sum(_leaky(xr2B + xl2A) * att2, axis=1, keepdims=True)
    a_self2 = jnp.sum(_leaky(xr2B + xl2B) * att2, axis=1, keepdims=True)
    m2 = jnp.maximum(a_pair2, a_self2)
    wp2 = 42.0 * jnp.exp(a_pair2 - m2)
    ws2 = jnp.exp(a_self2 - m2)
    inv2 = 1.0 / (wp2 + ws2 + 1e-16)
    bias2 = bias2_ref[...]
    h2A = xl2A + bias2
    h2B = (wp2 * inv2) * xl2A + (ws2 * inv2) * xl2B + bias2

    pool = pool_ref[...]
    gA = _dot(pool, h2A)
    gB = _dot(pool, h2B)

    w1 = w1_ref[...]; b1 = b1_ref[...]
    w2 = w2_ref[...]; b2 = b2_ref[...]
    w3 = w3_ref[...]; b3 = b3_ref[...]

    def mlp(g):
        t = jnp.maximum(_dot(g, w1) + b1, 0.0)
        t = jnp.maximum(_dot(t, w2) + b2, 0.0)
        return jnp.tanh(_dot(t, w3) + b3)

    outa_ref[...] = mlp(gA)
    outb_ref[...] = mlp(gB)


def kernel(state24, Wl1, bl1, Wr1, br1, att1, bias1,
           Wl2, bl2, Wr2, br2, att2, bias2,
           W1, b1, W2, b2, W3, b3):
    f32 = jnp.float32
    laser = (state24[:, :20] / MAX_RANGE)[..., None]
    angle = jnp.broadcast_to(jnp.asarray(_ANGLE_np)[None], (B, 20, 2))
    zeros4 = jnp.zeros((B, 20, 4), f32)
    sector = jnp.concatenate([laser, angle, zeros4], axis=-1)
    robot = jnp.concatenate([jnp.zeros((B, 1, 3), f32),
                             state24[:, 20:][:, None, :]], axis=-1)
    x = jnp.concatenate([sector, robot], axis=1).reshape(B * N_NODES, 7)
    xA, xB = x[:NHALF], x[NHALF:]

    sel = jnp.asarray(_E1_np.T)
    a1 = sel * att1.reshape(-1)[:, None]

    row_spec = pl.BlockSpec((ROWS, 7), lambda i: (i, 0))
    full = lambda shape: pl.BlockSpec(shape, lambda i: (0, 0))
    out_spec = pl.BlockSpec((TILE, 2), lambda i: (i, 0))

    outA, outB = pl.pallas_call(
        _actor_kernel,
        grid=(GRID,),
        in_specs=[
            row_spec, row_spec,
            full((7, 256)), full((1, 256)), full((7, 256)), full((1, 256)),
            full((256, 4)), full((4, 256)), full((1, 256)),
            full((256, 64)), full((1, 64)), full((256, 64)), full((1, 64)),
            full((1, 64)), full((1, 64)),
            full((TILE, ROWS)),
            full((64, 256)), full((1, 256)), full((256, 256)), full((1, 256)),
            full((256, 2)), full((1, 2)),
        ],
        out_specs=[out_spec, out_spec],
        out_shape=[jax.ShapeDtypeStruct((HALF, 2), f32),
                   jax.ShapeDtypeStruct((HALF, 2), f32)],
    )(
        xA, xB,
        Wl1, bl1.reshape(1, 256), Wr1, br1.reshape(1, 256),
        a1, jnp.asarray(_E1_np), bias1.reshape(1, 256),
        Wl2, bl2.reshape(1, 64), Wr2, br2.reshape(1, 64),
        att2.reshape(1, 64), bias2.reshape(1, 64),
        jnp.asarray(_POOL_np),
        W1, b1.reshape(1, 256), W2, b2.reshape(1, 256),
        W3, b3.reshape(1, 2),
    )
    return jnp.concatenate([outA, outB], axis=0)


# speedup vs baseline: 1877.7392x; 3.3707x over previous
import jax
import jax.numpy as jnp
import numpy as np
from jax.experimental import pallas as pl

N_NODES = 21
B = 1024
HALF = B // 2
NHALF = HALF * N_NODES
MAX_RANGE = 10.0
TILE = 128
ROWS = TILE * N_NODES
GRID = HALF // TILE

_HIGH = jax.lax.Precision.DEFAULT


def _angle_feat_np():
    bound = np.linspace(-np.pi / 2 - 0.03, np.pi / 2, 21)[:-1]
    angles = bound + np.pi / 20
    return np.stack([np.sin(angles), np.cos(angles)], axis=1).astype(np.float32)


def _feature_map_np():
    S = np.zeros((24, 147), dtype=np.float32)
    C = np.zeros((147,), dtype=np.float32)
    ang = _angle_feat_np()
    for n in range(20):
        S[n, 7 * n] = 1.0 / MAX_RANGE
        C[7 * n + 1] = ang[n, 0]
        C[7 * n + 2] = ang[n, 1]
    for j in range(4):
        S[20 + j, 143 + j] = 1.0
    return S, C


_S_np, _C_np = _feature_map_np()


def _head_selector(att_shape_heads, ch):
    c = np.arange(att_shape_heads * ch)
    m = (c[:, None] // ch == np.arange(att_shape_heads)[None, :]).astype(np.float32)
    return m


_E1_np = _head_selector(4, 64).T
_POOL_np = np.kron(np.eye(TILE, dtype=np.float32),
                   np.full((1, N_NODES), 1.0 / N_NODES, dtype=np.float32))


def _dot(a, b):
    return jnp.dot(a, b, precision=_HIGH, preferred_element_type=jnp.float32)


def _leaky(x):
    return jnp.where(x > 0, x, 0.2 * x)


def _elu(x):
    return jnp.where(x > 0, x, jnp.exp(jnp.minimum(x, 0.0)) - 1.0)


def _actor_kernel(xa_ref, xb_ref,
                  wl1_ref, bl1_ref, wlr1_ref, blr1_ref, a1_ref, e1_ref, bias1_ref,
                  wl2_ref, bl2_ref, wr2_ref, br2_ref, att2_ref, bias2_ref,
                  pool_ref,
                  w1_ref, b1_ref, w2_ref, b2_ref, w3_ref, b3_ref,
                  outa_ref, outb_ref):
    xa = xa_ref[...]
    xb = xb_ref[...]

    wl1 = wl1_ref[...]
    bl1 = bl1_ref[...]
    xlA = _dot(xa, wl1) + bl1
    xlrB = _dot(xb, wlr1_ref[...]) + blr1_ref[...]
    xlB = xlrB[:, :256]
    xrB = xlrB[:, 256:]

    a1 = a1_ref[...]
    a_pair = _dot(_leaky(xrB + xlA), a1)
    a_self = _dot(_leaky(xrB + xlB), a1)
    m = jnp.maximum(a_pair, a_self)
    wp = 42.0 * jnp.exp(a_pair - m)
    ws = jnp.exp(a_self - m)
    inv = 1.0 / (wp + ws + 1e-16)
    e1 = e1_ref[...]
    cp = _dot(wp * inv, e1)
    cs = _dot(ws * inv, e1)
    bias1 = bias1_ref[...]
    h1A = _elu(xlA + bias1)
    h1B = _elu(cp * xlA + cs * xlB + bias1)

    wl2 = wl2_ref[...]
    bl2 = bl2_ref[...]
    xl2A = _dot(h1A, wl2) + bl2
    xl2B = _dot(h1B, wl2) + bl2
    xr2B = _dot(h1B, wr2_ref[...]) + br2_ref[...]

    att2 = att2_ref[...]
    a_pair2 = jnp.sum(_leaky(xr2B + xl2A) * att2, axis=1, keepdims=True)
    a_self2 = jnp.sum(_leaky(xr2B + xl2B) * att2, axis=1, keepdims=True)
    m2 = jnp.maximum(a_pair2, a_self2)
    wp2 = 42.0 * jnp.exp(a_pair2 - m2)
    ws2 = jnp.exp(a_self2 - m2)
    inv2 = 1.0 / (wp2 + ws2 + 1e-16)
    bias2 = bias2_ref[...]
    h2A = xl2A + bias2
    h2B = (wp2 * inv2) * xl2A + (ws2 * inv2) * xl2B + bias2

    pool = pool_ref[...]
    gA = _dot(pool, h2A)
    gB = _dot(pool, h2B)

    w1 = w1_ref[...]; b1 = b1_ref[...]
    w2 = w2_ref[...]; b2 = b2_ref[...]
    w3 = w3_ref[...]; b3 = b3_ref[...]

    def mlp(g):
        t = jnp.maximum(_dot(g, w1) + b1, 0.0)
        t = jnp.maximum(_dot(t, w2) + b2, 0.0)
        return jnp.tanh(_dot(t, w3) + b3)

    outa_ref[...] = mlp(gA)
    outb_ref[...] = mlp(gB)


def kernel(state24, Wl1, bl1, Wr1, br1, att1, bias1,
           Wl2, bl2, Wr2, br2, att2, bias2,
           W1, b1, W2, b2, W3, b3):
    f32 = jnp.float32
    x = (jnp.dot(state24, jnp.asarray(_S_np)) + jnp.asarray(_C_np)
         ).reshape(B * N_NODES, 7)
    xA, xB = x[:NHALF], x[NHALF:]

    sel = jnp.asarray(_E1_np.T)
    a1 = sel * att1.reshape(-1)[:, None]

    row_spec = pl.BlockSpec((ROWS, 7), lambda i: (i, 0))
    full = lambda shape: pl.BlockSpec(shape, lambda i: (0, 0))
    out_spec = pl.BlockSpec((TILE, 2), lambda i: (i, 0))

    outA, outB = pl.pallas_call(
        _actor_kernel,
        grid=(GRID,),
        in_specs=[
            row_spec, row_spec,
            full((7, 256)), full((1, 256)), full((7, 512)), full((1, 512)),
            full((256, 4)), full((4, 256)), full((1, 256)),
            full((256, 64)), full((1, 64)), full((256, 64)), full((1, 64)),
            full((1, 64)), full((1, 64)),
            full((TILE, ROWS)),
            full((64, 256)), full((1, 256)), full((256, 256)), full((1, 256)),
            full((256, 2)), full((1, 2)),
        ],
        out_specs=[out_spec, out_spec],
        out_shape=[jax.ShapeDtypeStruct((HALF, 2), f32),
                   jax.ShapeDtypeStruct((HALF, 2), f32)],
    )(
        xA, xB,
        Wl1, bl1.reshape(1, 256),
        jnp.concatenate([Wl1, Wr1], axis=1),
        jnp.concatenate([bl1, br1]).reshape(1, 512),
        a1, jnp.asarray(_E1_np), bias1.reshape(1, 256),
        Wl2, bl2.reshape(1, 64), Wr2, br2.reshape(1, 64),
        att2.reshape(1, 64), bias2.reshape(1, 64),
        jnp.asarray(_POOL_np),
        W1, b1.reshape(1, 256), W2, b2.reshape(1, 256),
        W3, b3.reshape(1, 2),
    )
    return jnp.concatenate([outA, outB], axis=0)


# speedup vs baseline: 1971.5060x; 1.0499x over previous
import jax
import jax.numpy as jnp
import numpy as np
from jax.experimental import pallas as pl

N_NODES = 21
B = 1024
HALF = B // 2
NHALF = HALF * N_NODES
MAX_RANGE = 10.0
TILE = 128
ROWS = TILE * N_NODES
GRID = HALF // TILE

_HIGH = jax.lax.Precision.DEFAULT


def _angle_feat_np():
    bound = np.linspace(-np.pi / 2 - 0.03, np.pi / 2, 21)[:-1]
    angles = bound + np.pi / 20
    return np.stack([np.sin(angles), np.cos(angles)], axis=1).astype(np.float32)


def _feature_map_np():
    S = np.zeros((24, 147), dtype=np.float32)
    C = np.zeros((147,), dtype=np.float32)
    ang = _angle_feat_np()
    for n in range(20):
        S[n, 7 * n] = 1.0 / MAX_RANGE
        C[7 * n + 1] = ang[n, 0]
        C[7 * n + 2] = ang[n, 1]
    for j in range(4):
        S[20 + j, 143 + j] = 1.0
    return S, C


_S_np, _C_np = _feature_map_np()


def _head_selector(att_shape_heads, ch):
    c = np.arange(att_shape_heads * ch)
    m = (c[:, None] // ch == np.arange(att_shape_heads)[None, :]).astype(np.float32)
    return m


_E1_np = _head_selector(4, 64).T
_POOL_np = np.kron(np.eye(TILE, dtype=np.float32),
                   np.full((1, N_NODES), 1.0 / N_NODES, dtype=np.float32))


def _dot(a, b):
    return jnp.dot(a, b, precision=_HIGH, preferred_element_type=jnp.float32)


def _leaky(x):
    return jnp.maximum(x, 0.2 * x)


def _elu(x):
    return jnp.maximum(x, jnp.exp(jnp.minimum(x, 0.0)) - 1.0)


def _actor_kernel(xa_ref, xb_ref,
                  wl1_ref, bl1_ref, wlr1_ref, blr1_ref, a1_ref, e1_ref, bias1_ref,
                  wl2_ref, bl2_ref, wr2_ref, br2_ref, att2_ref, bias2_ref,
                  pool_ref,
                  w1_ref, b1_ref, w2_ref, b2_ref, w3_ref, b3_ref,
                  outa_ref, outb_ref):
    xa = xa_ref[...]
    xb = xb_ref[...]

    wl1 = wl1_ref[...]
    bl1 = bl1_ref[...]
    xlA = _dot(xa, wl1) + bl1
    xlrB = _dot(xb, wlr1_ref[...]) + blr1_ref[...]
    xlB = xlrB[:, :256]
    xrB = xlrB[:, 256:]

    a1 = a1_ref[...]
    a_pair = _dot(_leaky(xrB + xlA), a1)
    a_self = _dot(_leaky(xrB + xlB), a1)
    m = jnp.maximum(a_pair, a_self)
    wp = 42.0 * jnp.exp(a_pair - m)
    ws = jnp.exp(a_self - m)
    cs = _dot(ws / (wp + ws + 1e-16), e1_ref[...])
    bias1 = bias1_ref[...]
    h1A = _elu(xlA + bias1)
    h1B = _elu(xlA + cs * (xlB - xlA) + bias1)

    wl2 = wl2_ref[...]
    bl2 = bl2_ref[...]
    xl2A = _dot(h1A, wl2) + bl2
    xl2B = _dot(h1B, wl2) + bl2
    xr2B = _dot(h1B, wr2_ref[...]) + br2_ref[...]

    att2 = att2_ref[...]
    a_pair2 = jnp.sum(_leaky(xr2B + xl2A) * att2, axis=1, keepdims=True)
    a_self2 = jnp.sum(_leaky(xr2B + xl2B) * att2, axis=1, keepdims=True)
    m2 = jnp.maximum(a_pair2, a_self2)
    wp2 = 42.0 * jnp.exp(a_pair2 - m2)
    ws2 = jnp.exp(a_self2 - m2)
    cs2 = ws2 / (wp2 + ws2 + 1e-16)
    bias2 = bias2_ref[...]
    h2A = xl2A + bias2
    h2B = xl2A + cs2 * (xl2B - xl2A) + bias2

    pool = pool_ref[...]
    gA = _dot(pool, h2A)
    gB = _dot(pool, h2B)

    g = jnp.concatenate([gA, gB], axis=0)
    t = jnp.maximum(_dot(g, w1_ref[...]) + b1_ref[...], 0.0)
    t = jnp.maximum(_dot(t, w2_ref[...]) + b2_ref[...], 0.0)
    o = jnp.tanh(_dot(t, w3_ref[...]) + b3_ref[...])
    outa_ref[...] = o[:TILE]
    outb_ref[...] = o[TILE:]


def kernel(state24, Wl1, bl1, Wr1, br1, att1, bias1,
           Wl2, bl2, Wr2, br2, att2, bias2,
           W1, b1, W2, b2, W3, b3):
    f32 = jnp.float32
    x = (jnp.dot(state24, jnp.asarray(_S_np)) + jnp.asarray(_C_np)
         ).reshape(B * N_NODES, 7)
    xA, xB = x[:NHALF], x[NHALF:]

    sel = jnp.asarray(_E1_np.T)
    a1 = sel * att1.reshape(-1)[:, None]

    row_spec = pl.BlockSpec((ROWS, 7), lambda i: (i, 0))
    full = lambda shape: pl.BlockSpec(shape, lambda i: (0, 0))
    out_spec = pl.BlockSpec((TILE, 2), lambda i: (i, 0))

    outA, outB = pl.pallas_call(
        _actor_kernel,
        grid=(GRID,),
        in_specs=[
            row_spec, row_spec,
            full((7, 256)), full((1, 256)), full((7, 512)), full((1, 512)),
            full((256, 4)), full((4, 256)), full((1, 256)),
            full((256, 64)), full((1, 64)), full((256, 64)), full((1, 64)),
            full((1, 64)), full((1, 64)),
            full((TILE, ROWS)),
            full((64, 256)), full((1, 256)), full((256, 256)), full((1, 256)),
            full((256, 2)), full((1, 2)),
        ],
        out_specs=[out_spec, out_spec],
        out_shape=[jax.ShapeDtypeStruct((HALF, 2), f32),
                   jax.ShapeDtypeStruct((HALF, 2), f32)],
    )(
        xA, xB,
        Wl1, bl1.reshape(1, 256),
        jnp.concatenate([Wl1, Wr1], axis=1),
        jnp.concatenate([bl1, br1]).reshape(1, 512),
        a1, jnp.asarray(_E1_np), bias1.reshape(1, 256),
        Wl2, bl2.reshape(1, 64), Wr2, br2.reshape(1, 64),
        att2.reshape(1, 64), bias2.reshape(1, 64),
        jnp.asarray(_POOL_np),
        W1, b1.reshape(1, 256), W2, b2.reshape(1, 256),
        W3, b3.reshape(1, 2),
    )
    return jnp.concatenate([outA, outB], axis=0)


# speedup vs baseline: 2113.9189x; 1.0722x over previous
import jax
import jax.numpy as jnp
import numpy as np
from jax.experimental import pallas as pl

N_NODES = 21
B = 1024
HALF = B // 2
NHALF = HALF * N_NODES
MAX_RANGE = 10.0
TILE = 128
ROWS = TILE * N_NODES
GRID = HALF // TILE

_PREC = jax.lax.Precision.DEFAULT


def _angle_feat_np():
    bound = np.linspace(-np.pi / 2 - 0.03, np.pi / 2, 21)[:-1]
    angles = bound + np.pi / 20
    return np.stack([np.sin(angles), np.cos(angles)], axis=1).astype(np.float32)


def _feature_map_np():
    S = np.zeros((N_NODES, 24, 7), dtype=np.float32)
    C = np.zeros((N_NODES, 7), dtype=np.float32)
    ang = _angle_feat_np()
    for n in range(20):
        S[n, n, 0] = 1.0 / MAX_RANGE
        C[n, 1] = ang[n, 0]
        C[n, 2] = ang[n, 1]
    for j in range(4):
        S[20, 20 + j, 3 + j] = 1.0
    return S, C


_S_np, _C_np = _feature_map_np()

_E1_np = (np.arange(256)[None, :] // 64 == np.arange(4)[:, None]).astype(np.float32)
_POOL_np = np.kron(np.full((1, N_NODES), 1.0 / N_NODES, dtype=np.float32),
                   np.eye(TILE, dtype=np.float32))


def _dot(a, b):
    return jnp.dot(a, b, precision=_PREC, preferred_element_type=jnp.float32)


def _leaky(x):
    return jnp.maximum(x, 0.2 * x)


def _elu(x):
    return jnp.maximum(x, jnp.exp(jnp.minimum(x, 0.0)) - 1.0)


def _actor_kernel(sa_ref, sb_ref,
                  wna_ref, cna_ref, wnb_ref, cnb_ref,
                  a1_ref, e1_ref, bias1_ref,
                  wl2_ref, bl2_ref, wr2_ref, br2_ref, att2_ref, bias2_ref,
                  pool_ref,
                  w1_ref, b1_ref, w2_ref, b2_ref, w3_ref, b3_ref,
                  outa_ref, outb_ref):
    sa = sa_ref[...]
    sb = sb_ref[...]

    wna = wna_ref[...]
    cna = cna_ref[...]
    wnb = wnb_ref[...]
    cnb = cnb_ref[...]
    xlA = jnp.concatenate(
        [_dot(sa, wna[24 * n:24 * n + 24]) + cna[n:n + 1] for n in range(N_NODES)],
        axis=0)
    xlrB = jnp.concatenate(
        [_dot(sb, wnb[24 * n:24 * n + 24]) + cnb[n:n + 1] for n in range(N_NODES)],
        axis=0)
    xlB = xlrB[:, :256]
    xrB = xlrB[:, 256:]

    a1 = a1_ref[...]
    a_pair = _dot(_leaky(xrB + xlA), a1)
    a_self = _dot(_leaky(xrB + xlB), a1)
    m = jnp.maximum(a_pair, a_self)
    wp = 42.0 * jnp.exp(a_pair - m)
    ws = jnp.exp(a_self - m)
    cs = _dot(ws / (wp + ws + 1e-16), e1_ref[...])
    bias1 = bias1_ref[...]
    h1A = _elu(xlA + bias1)
    h1B = _elu(xlA + cs * (xlB - xlA) + bias1)

    wl2 = wl2_ref[...]
    bl2 = bl2_ref[...]
    xl2A = _dot(h1A, wl2) + bl2
    xl2B = _dot(h1B, wl2) + bl2
    xr2B = _dot(h1B, wr2_ref[...]) + br2_ref[...]

    att2 = att2_ref[...]
    a_pair2 = jnp.sum(_leaky(xr2B + xl2A) * att2, axis=1, keepdims=True)
    a_self2 = jnp.sum(_leaky(xr2B + xl2B) * att2, axis=1, keepdims=True)
    m2 = jnp.maximum(a_pair2, a_self2)
    wp2 = 42.0 * jnp.exp(a_pair2 - m2)
    ws2 = jnp.exp(a_self2 - m2)
    cs2 = ws2 / (wp2 + ws2 + 1e-16)
    bias2 = bias2_ref[...]
    h2A = xl2A + bias2
    h2B = xl2A + cs2 * (xl2B - xl2A) + bias2

    pool = pool_ref[...]
    gA = _dot(pool, h2A)
    gB = _dot(pool, h2B)

    g = jnp.concatenate([gA, gB], axis=0)
    t = jnp.maximum(_dot(g, w1_ref[...]) + b1_ref[...], 0.0)
    t = jnp.maximum(_dot(t, w2_ref[...]) + b2_ref[...], 0.0)
    o = jnp.tanh(_dot(t, w3_ref[...]) + b3_ref[...])
    outa_ref[...] = o[:TILE]
    outb_ref[...] = o[TILE:]


def kernel(state24, Wl1, bl1, Wr1, br1, att1, bias1,
           Wl2, bl2, Wr2, br2, att2, bias2,
           W1, b1, W2, b2, W3, b3):
    f32 = jnp.float32
    S = jnp.asarray(_S_np)
    C = jnp.asarray(_C_np)
    Wlr1 = jnp.concatenate([Wl1, Wr1], axis=1)
    blr1 = jnp.concatenate([bl1, br1])
    wna = jnp.einsum('nkf,fc->nkc', S, Wl1).reshape(N_NODES * 24, 256)
    cna = jnp.dot(C, Wl1) + bl1
    wnb = jnp.einsum('nkf,fc->nkc', S, Wlr1).reshape(N_NODES * 24, 512)
    cnb = jnp.dot(C, Wlr1) + blr1

    sel = jnp.asarray(_E1_np.T)
    a1 = sel * att1.reshape(-1)[:, None]

    state_spec = pl.BlockSpec((TILE, 24), lambda i: (i, 0))
    full = lambda shape: pl.BlockSpec(shape, lambda i: (0, 0))
    out_spec = pl.BlockSpec((TILE, 2), lambda i: (i, 0))

    outA, outB = pl.pallas_call(
        _actor_kernel,
        grid=(GRID,),
        in_specs=[
            state_spec, state_spec,
            full((N_NODES * 24, 256)), full((N_NODES, 256)),
            full((N_NODES * 24, 512)), full((N_NODES, 512)),
            full((256, 4)), full((4, 256)), full((1, 256)),
            full((256, 64)), full((1, 64)), full((256, 64)), full((1, 64)),
            full((1, 64)), full((1, 64)),
            full((TILE, ROWS)),
            full((64, 256)), full((1, 256)), full((256, 256)), full((1, 256)),
            full((256, 2)), full((1, 2)),
        ],
        out_specs=[out_spec, out_spec],
        out_shape=[jax.ShapeDtypeStruct((HALF, 2), f32),
                   jax.ShapeDtypeStruct((HALF, 2), f32)],
    )(
        state24[:HALF], state24[HALF:],
        wna, cna, wnb, cnb,
        a1, jnp.asarray(_E1_np), bias1.reshape(1, 256),
        Wl2, bl2.reshape(1, 64), Wr2, br2.reshape(1, 64),
        att2.reshape(1, 64), bias2.reshape(1, 64),
        jnp.asarray(_POOL_np),
        W1, b1.reshape(1, 256), W2, b2.reshape(1, 256),
        W3, b3.reshape(1, 2),
    )
    return jnp.concatenate([outA, outB], axis=0)


# speedup vs baseline: 2691.0573x; 1.2730x over previous
import jax
import jax.numpy as jnp
import numpy as np
from jax.experimental import pallas as pl
from jax.experimental.pallas import tpu as pltpu

N_NODES = 21
B = 1024
HALF = B // 2
MAX_RANGE = 10.0
TILE = 128
ROWS = TILE * N_NODES
GRID = HALF // TILE

_PREC = jax.lax.Precision.DEFAULT


def _angle_feat_np():
    bound = np.linspace(-np.pi / 2 - 0.03, np.pi / 2, 21)[:-1]
    angles = bound + np.pi / 20
    return np.stack([np.sin(angles), np.cos(angles)], axis=1).astype(np.float32)


def _feature_map_np():
    S = np.zeros((N_NODES, 24, 7), dtype=np.float32)
    C = np.zeros((N_NODES, 7), dtype=np.float32)
    ang = _angle_feat_np()
    for n in range(20):
        S[n, n, 0] = 1.0 / MAX_RANGE
        C[n, 1] = ang[n, 0]
        C[n, 2] = ang[n, 1]
    for j in range(4):
        S[20, 20 + j, 3 + j] = 1.0
    return S.reshape(N_NODES * 24, 7), C


_S_np, _C_np = _feature_map_np()

_E1_np = (np.arange(256)[None, :] // 64 == np.arange(4)[:, None]).astype(np.float32)
_POOL_np = np.kron(np.full((1, N_NODES), 1.0 / N_NODES, dtype=np.float32),
                   np.eye(TILE, dtype=np.float32))


def _dot(a, b):
    return jnp.dot(a, b, precision=_PREC, preferred_element_type=jnp.float32)


def _leaky(x):
    return jnp.maximum(x, 0.2 * x)


def _elu(x):
    return jnp.maximum(x, jnp.exp(jnp.minimum(x, 0.0)) - 1.0)


def _actor_kernel(sa_ref, sb_ref,
                  s_ref, c_ref, sel_ref, e1_ref,
                  wl1_ref, bl1_ref, wr1_ref, br1_ref, att1_ref, bias1_ref,
                  wl2_ref, bl2_ref, wr2_ref, br2_ref, att2_ref, bias2_ref,
                  pool_ref,
                  w1_ref, b1_ref, w2_ref, b2_ref, w3_ref, b3_ref,
                  out_ref,
                  wna_scr, cna_scr, wnb_scr, cnb_scr, a1_scr):

    @pl.when(pl.program_id(0) == 0)
    def _fold():
        wl1 = wl1_ref[...]
        wlr = jnp.concatenate([wl1, wr1_ref[...]], axis=1)
        blr = jnp.concatenate([bl1_ref[...], br1_ref[...]], axis=1)
        s = s_ref[...]
        c = c_ref[...]
        wna_scr[...] = _dot(s, wl1)
        cna_scr[...] = _dot(c, wl1) + bl1_ref[...]
        wnb_scr[...] = _dot(s, wlr)
        cnb_scr[...] = _dot(c, wlr) + blr
        a1_scr[...] = sel_ref[...] * att1_ref[...]

    sa = sa_ref[...]
    sb = sb_ref[...]

    wna = wna_scr[...]
    cna = cna_scr[...]
    wnb = wnb_scr[...]
    cnb = cnb_scr[...]
    xlA = jnp.concatenate(
        [_dot(sa, wna[24 * n:24 * n + 24]) + cna[n:n + 1] for n in range(N_NODES)],
        axis=0)
    xlrB = jnp.concatenate(
        [_dot(sb, wnb[24 * n:24 * n + 24]) + cnb[n:n + 1] for n in range(N_NODES)],
        axis=0)
    xlB = xlrB[:, :256]
    xrB = xlrB[:, 256:]

    a1 = a1_scr[...]
    a_pair = _dot(_leaky(xrB + xlA), a1)
    a_self = _dot(_leaky(xrB + xlB), a1)
    m = jnp.maximum(a_pair, a_self)
    wp = 42.0 * jnp.exp(a_pair - m)
    ws = jnp.exp(a_self - m)
    cs = _dot(ws / (wp + ws + 1e-16), e1_ref[...])
    bias1 = bias1_ref[...]
    h1A = _elu(xlA + bias1)
    h1B = _elu(xlA + cs * (xlB - xlA) + bias1)

    wl2 = wl2_ref[...]
    bl2 = bl2_ref[...]
    xl2A = _dot(h1A, wl2) + bl2
    xl2B = _dot(h1B, wl2) + bl2
    xr2B = _dot(h1B, wr2_ref[...]) + br2_ref[...]

    att2 = att2_ref[...]
    a_pair2 = jnp.sum(_leaky(xr2B + xl2A) * att2, axis=1, keepdims=True)
    a_self2 = jnp.sum(_leaky(xr2B + xl2B) * att2, axis=1, keepdims=True)
    m2 = jnp.maximum(a_pair2, a_self2)
    wp2 = 42.0 * jnp.exp(a_pair2 - m2)
    ws2 = jnp.exp(a_self2 - m2)
    cs2 = ws2 / (wp2 + ws2 + 1e-16)
    bias2 = bias2_ref[...]
    h2A = xl2A + bias2
    h2B = xl2A + cs2 * (xl2B - xl2A) + bias2

    pool = pool_ref[...]
    gA = _dot(pool, h2A)
    gB = _dot(pool, h2B)

    g = jnp.concatenate([gA, gB], axis=0)
    t = jnp.maximum(_dot(g, w1_ref[...]) + b1_ref[...], 0.0)
    t = jnp.maximum(_dot(t, w2_ref[...]) + b2_ref[...], 0.0)
    o = jnp.tanh(_dot(t, w3_ref[...]) + b3_ref[...])
    out_ref[0] = o[:TILE]
    out_ref[1] = o[TILE:]


def kernel(state24, Wl1, bl1, Wr1, br1, att1, bias1,
           Wl2, bl2, Wr2, br2, att2, bias2,
           W1, b1, W2, b2, W3, b3):
    f32 = jnp.float32
    sa_spec = pl.BlockSpec((TILE, 24), lambda i: (i, 0))
    sb_spec = pl.BlockSpec((TILE, 24), lambda i: (i + GRID, 0))
    full = lambda shape: pl.BlockSpec(shape, lambda i: tuple(0 for _ in shape))
    out_spec = pl.BlockSpec((2, TILE, 2), lambda i: (0, i, 0))

    out = pl.pallas_call(
        _actor_kernel,
        grid=(GRID,),
        in_specs=[
            sa_spec, sb_spec,
            full((N_NODES * 24, 7)), full((N_NODES, 7)),
            full((256, 4)), full((4, 256)),
            full((7, 256)), full((1, 256)), full((7, 256)), full((1, 256)),
            full((256, 1)), full((1, 256)),
            full((256, 64)), full((1, 64)), full((256, 64)), full((1, 64)),
            full((1, 64)), full((1, 64)),
            full((TILE, ROWS)),
            full((64, 256)), full((1, 256)), full((256, 256)), full((1, 256)),
            full((256, 2)), full((1, 2)),
        ],
        out_specs=out_spec,
        out_shape=jax.ShapeDtypeStruct((2, HALF, 2), f32),
        scratch_shapes=[
            pltpu.VMEM((N_NODES * 24, 256), f32),
            pltpu.VMEM((N_NODES, 256), f32),
            pltpu.VMEM((N_NODES * 24, 512), f32),
            pltpu.VMEM((N_NODES, 512), f32),
            pltpu.VMEM((256, 4), f32),
        ],
    )(
        state24, state24,
        jnp.asarray(_S_np), jnp.asarray(_C_np),
        jnp.asarray(_E1_np.T), jnp.asarray(_E1_np),
        Wl1, bl1.reshape(1, 256), Wr1, br1.reshape(1, 256),
        att1.reshape(256, 1), bias1.reshape(1, 256),
        Wl2, bl2.reshape(1, 64), Wr2, br2.reshape(1, 64),
        att2.reshape(1, 64), bias2.reshape(1, 64),
        jnp.asarray(_POOL_np),
        W1, b1.reshape(1, 256), W2, b2.reshape(1, 256),
        W3, b3.reshape(1, 2),
    )
    return out.reshape(B, 2)
